# baseline (speedup 1.0000x reference)
import functools

import jax
import jax.numpy as jnp
from jax import lax
from jax.experimental import pallas as pl
from jax.experimental.pallas import tpu as pltpu
from jax.experimental.pallas import tpu_sc as plsc

_K = 19
_KP = 32
_R = 256


def _round_up(x, m):
    return (x + m - 1) // m * m


def _knn_body(n, npad, r, pos_rows_ref, post_ref, idx_ref, d2_ref, colf_ref):
    nchunk = npad // 128
    i = pl.program_id(0)
    pr = pos_rows_ref[...]
    pc = post_ref[...]
    g = jnp.dot(pr, pc, preferred_element_type=jnp.float32)
    sq_r = (pr[:, 0:1] * pr[:, 0:1] + pr[:, 2:3] * pr[:, 2:3]) \
        + pr[:, 1:2] * pr[:, 1:2]
    sq_c = (pc[0:1, :] * pc[0:1, :] + pc[2:3, :] * pc[2:3, :]) \
        + pc[1:2, :] * pc[1:2, :]
    d2 = jnp.maximum(sq_r + sq_c - 2.0 * g, 0.0)
    col = lax.broadcasted_iota(jnp.int32, (r, npad), 1)
    row = i * r + lax.broadcasted_iota(jnp.int32, (r, npad), 0)
    inf = jnp.float32(jnp.inf)
    big = jnp.float32(3.0e7)
    d2 = jnp.where((col == row) | (col >= n), inf, d2)
    colf = col.astype(jnp.float32)
    for c in range(nchunk):
        d2_ref[c] = d2[:, c * 128:(c + 1) * 128]
        colf_ref[c] = colf[:, c * 128:(c + 1) * 128]

    lane = lax.broadcasted_iota(jnp.int32, (r, _KP), 1)

    def body(k, carry):
        idxs, prevf = carry

        def sweep1(c, mvec):
            blkm = jnp.where(colf_ref[c] == prevf, inf, d2_ref[c])
            d2_ref[c] = blkm
            return jnp.minimum(mvec, blkm)

        mvec = lax.fori_loop(0, nchunk, sweep1,
                             jnp.full((r, 128), inf, jnp.float32))
        m = jnp.min(mvec, axis=1, keepdims=True)

        def sweep2(c, ivec):
            return jnp.minimum(
                ivec, jnp.where(d2_ref[c] == m, colf_ref[c], big))

        ivec = lax.fori_loop(0, nchunk, sweep2,
                             jnp.full((r, 128), big, jnp.float32))
        idxf = jnp.min(ivec, axis=1, keepdims=True)
        idxs = jnp.where(lane == k, idxf.astype(jnp.int32), idxs)
        return idxs, idxf

    init = (jnp.zeros((r, _KP), jnp.int32), jnp.full((r, 1), -1.0, jnp.float32))
    idx_ref[...] = lax.fori_loop(0, _K, body, init)[0]


def _edge_body(k, nc, bpw, npad,
               px_h, py_h, pz_h, src_h, vx_h, vy_h, vz_h, s2_h,
               px_v, py_v, pz_v, src_v, vx_v, vy_v, vz_v, s2_v):
    wid = lax.axis_index("s") * nc + lax.axis_index("c")
    base = wid * bpw
    pltpu.sync_copy(px_h, px_v)
    pltpu.sync_copy(py_h, py_v)
    pltpu.sync_copy(pz_h, pz_v)
    pltpu.sync_copy(src_h.at[pl.ds(base, bpw)], src_v)

    def body(j, carry):
        sl = pl.ds(j * 16, 16)
        sidx = src_v[sl]
        e = jnp.full((16,), base + j * 16, jnp.int32) + lax.iota(jnp.int32, 16)
        didx = lax.div(e, jnp.full((16,), k, jnp.int32))
        xs = plsc.load_gather(px_v, [sidx])
        ys = plsc.load_gather(py_v, [sidx])
        zs = plsc.load_gather(pz_v, [sidx])
        xd = plsc.load_gather(px_v, [didx])
        yd = plsc.load_gather(py_v, [didx])
        zd = plsc.load_gather(pz_v, [didx])
        dx = xs - xd
        dy = ys - yd
        dz = zs - zd
        vx_v[sl] = dx
        vy_v[sl] = dy
        vz_v[sl] = dz
        s2_v[sl] = (dx * dx + dz * dz) + dy * dy
        return carry

    lax.fori_loop(0, bpw // 16, body, 0)
    pltpu.sync_copy(vx_v, vx_h.at[pl.ds(base, bpw)])
    pltpu.sync_copy(vy_v, vy_h.at[pl.ds(base, bpw)])
    pltpu.sync_copy(vz_v, vz_h.at[pl.ds(base, bpw)])
    pltpu.sync_copy(s2_v, s2_h.at[pl.ds(base, bpw)])


def _len_body(s2_ref, len_ref):
    len_ref[...] = jnp.sqrt(s2_ref[...] + 1e-12)


@jax.jit
def kernel(pos):
    n = pos.shape[0]
    npad = _round_up(max(n, 128), 128)
    r = min(_R, _round_up(n, 8))
    nrow = _round_up(n, r)

    pos_rows = jnp.pad(pos, ((0, nrow - n), (0, 5)))
    post = jnp.pad(pos, ((0, npad - n), (0, 5))).T

    idxs = pl.pallas_call(
        functools.partial(_knn_body, n, npad, r),
        grid=(nrow // r,),
        in_specs=[
            pl.BlockSpec((r, 8), lambda i: (i, 0)),
            pl.BlockSpec((8, npad), lambda i: (0, 0)),
        ],
        out_specs=pl.BlockSpec((r, _KP), lambda i: (i, 0)),
        out_shape=jax.ShapeDtypeStruct((nrow, _KP), jnp.int32),
        scratch_shapes=[pltpu.VMEM((npad // 128, r, 128), jnp.float32),
                        pltpu.VMEM((npad // 128, r, 128), jnp.float32)],
    )(pos_rows, post)

    src = idxs[:n, :_K].reshape(-1)
    nk = n * _K

    info = plsc.get_sparse_core_info()
    nc, ns = info.num_cores, info.num_subcores
    nw = nc * ns
    bp = _round_up(nk, 16 * nw)
    bpw = bp // nw

    src_pad = jnp.pad(src, (0, bp - nk))
    pos_cpad = jnp.pad(pos, ((0, npad - n), (0, 0)))
    px = pos_cpad[:, 0]
    py = pos_cpad[:, 1]
    pz = pos_cpad[:, 2]

    f32 = jnp.float32
    vec_t = jax.ShapeDtypeStruct((bp,), f32)
    vx, vy, vz, s2 = pl.kernel(
        functools.partial(_edge_body, _K, nc, bpw, npad),
        out_type=(vec_t, vec_t, vec_t, vec_t),
        mesh=plsc.VectorSubcoreMesh(core_axis_name="c", subcore_axis_name="s"),
        compiler_params=pltpu.CompilerParams(needs_layout_passes=False),
        scratch_types=[
            pltpu.VMEM((npad,), f32),
            pltpu.VMEM((npad,), f32),
            pltpu.VMEM((npad,), f32),
            pltpu.VMEM((bpw,), jnp.int32),
            pltpu.VMEM((bpw,), f32),
            pltpu.VMEM((bpw,), f32),
            pltpu.VMEM((bpw,), f32),
            pltpu.VMEM((bpw,), f32),
        ],
    )(px, py, pz, src_pad)

    rows = bp // 128
    elen = pl.pallas_call(
        _len_body,
        grid=(1,),
        in_specs=[pl.BlockSpec((rows, 128), lambda i: (i, 0))],
        out_specs=pl.BlockSpec((rows, 128), lambda i: (i, 0)),
        out_shape=jax.ShapeDtypeStruct((rows, 128), jnp.float32),
    )(s2.reshape(rows, 128))

    dst = jnp.broadcast_to(
        jnp.arange(n, dtype=jnp.int32)[:, None], (n, _K)).reshape(-1)
    edge_index = jnp.stack([src, dst], axis=0)
    edge_vec = jnp.stack([vx[:nk], vy[:nk], vz[:nk]], axis=-1)
    edge_len = elen.reshape(-1)[:nk]
    return edge_index, edge_vec, edge_len


# speedup vs baseline: 2.0641x; 2.0641x over previous
import functools

import jax
import jax.numpy as jnp
from jax import lax
from jax.experimental import pallas as pl
from jax.experimental.pallas import tpu as pltpu
from jax.experimental.pallas import tpu_sc as plsc

_K = 19
_KP = 32
_R = 512


def _round_up(x, m):
    return (x + m - 1) // m * m


def _knn_body(n, npad, r, pos_rows_ref, post_ref, idx_ref, d2_ref):
    i = pl.program_id(0)
    pr = pos_rows_ref[...]
    pc = post_ref[...]
    g = jnp.dot(pr, pc, preferred_element_type=jnp.float32)
    sq_r = (pr[:, 0:1] * pr[:, 0:1] + pr[:, 2:3] * pr[:, 2:3]) \
        + pr[:, 1:2] * pr[:, 1:2]
    sq_c = (pc[0:1, :] * pc[0:1, :] + pc[2:3, :] * pc[2:3, :]) \
        + pc[1:2, :] * pc[1:2, :]
    d2 = jnp.maximum(sq_r + sq_c - 2.0 * g, 0.0)
    col = lax.broadcasted_iota(jnp.int32, (r, npad), 1)
    row = i * r + lax.broadcasted_iota(jnp.int32, (r, npad), 0)
    inf = jnp.float32(jnp.inf)
    d2 = jnp.where((col == row) | (col >= n), inf, d2)
    d2_ref[...] = d2

    lane = lax.broadcasted_iota(jnp.int32, (r, _KP), 1)
    colf = col.astype(jnp.float32)
    big = jnp.float32(npad)

    def body(k, carry):
        idxs, prevf = carry
        d2v = d2_ref[...]
        d2v = jnp.where(colf == prevf, inf, d2v)
        d2_ref[...] = d2v
        m = jnp.min(d2v, axis=1, keepdims=True)
        idxf = jnp.min(jnp.where(d2v == m, colf, big), axis=1, keepdims=True)
        return jnp.where(lane == k, idxf.astype(jnp.int32), idxs), idxf

    init = (jnp.zeros((r, _KP), jnp.int32), jnp.full((r, 1), -1.0, jnp.float32))
    idx_ref[...] = lax.fori_loop(0, _K, body, init)[0]


def _edge_body(k, nc, bpw, npad,
               px_h, py_h, pz_h, src_h, vx_h, vy_h, vz_h, s2_h,
               px_v, py_v, pz_v, src_v, vx_v, vy_v, vz_v, s2_v):
    wid = lax.axis_index("s") * nc + lax.axis_index("c")
    base = wid * bpw
    pltpu.sync_copy(px_h, px_v)
    pltpu.sync_copy(py_h, py_v)
    pltpu.sync_copy(pz_h, pz_v)
    pltpu.sync_copy(src_h.at[pl.ds(base, bpw)], src_v)

    def body(j, carry):
        sl = pl.ds(j * 16, 16)
        sidx = src_v[sl]
        e = jnp.full((16,), base + j * 16, jnp.int32) + lax.iota(jnp.int32, 16)
        didx = lax.div(e, jnp.full((16,), k, jnp.int32))
        xs = plsc.load_gather(px_v, [sidx])
        ys = plsc.load_gather(py_v, [sidx])
        zs = plsc.load_gather(pz_v, [sidx])
        xd = plsc.load_gather(px_v, [didx])
        yd = plsc.load_gather(py_v, [didx])
        zd = plsc.load_gather(pz_v, [didx])
        dx = xs - xd
        dy = ys - yd
        dz = zs - zd
        vx_v[sl] = dx
        vy_v[sl] = dy
        vz_v[sl] = dz
        s2_v[sl] = (dx * dx + dz * dz) + dy * dy
        return carry

    lax.fori_loop(0, bpw // 16, body, 0)
    pltpu.sync_copy(vx_v, vx_h.at[pl.ds(base, bpw)])
    pltpu.sync_copy(vy_v, vy_h.at[pl.ds(base, bpw)])
    pltpu.sync_copy(vz_v, vz_h.at[pl.ds(base, bpw)])
    pltpu.sync_copy(s2_v, s2_h.at[pl.ds(base, bpw)])


def _len_body(s2_ref, len_ref):
    len_ref[...] = jnp.sqrt(s2_ref[...] + 1e-12)


@jax.jit
def kernel(pos):
    n = pos.shape[0]
    npad = _round_up(max(n, 128), 128)
    r = min(_R, _round_up(n, 8))
    nrow = _round_up(n, r)

    pos_rows = jnp.pad(pos, ((0, nrow - n), (0, 5)))
    post = jnp.pad(pos, ((0, npad - n), (0, 5))).T

    idxs = pl.pallas_call(
        functools.partial(_knn_body, n, npad, r),
        grid=(nrow // r,),
        in_specs=[
            pl.BlockSpec((r, 8), lambda i: (i, 0)),
            pl.BlockSpec((8, npad), lambda i: (0, 0)),
        ],
        out_specs=pl.BlockSpec((r, _KP), lambda i: (i, 0)),
        out_shape=jax.ShapeDtypeStruct((nrow, _KP), jnp.int32),
        scratch_shapes=[pltpu.VMEM((r, npad), jnp.float32)],
    )(pos_rows, post)

    src = idxs[:n, :_K].reshape(-1)
    nk = n * _K

    info = plsc.get_sparse_core_info()
    nc, ns = info.num_cores, info.num_subcores
    nw = nc * ns
    bp = _round_up(nk, 16 * nw)
    bpw = bp // nw

    src_pad = jnp.pad(src, (0, bp - nk))
    pos_cpad = jnp.pad(pos, ((0, npad - n), (0, 0)))
    px = pos_cpad[:, 0]
    py = pos_cpad[:, 1]
    pz = pos_cpad[:, 2]

    f32 = jnp.float32
    vec_t = jax.ShapeDtypeStruct((bp,), f32)
    vx, vy, vz, s2 = pl.kernel(
        functools.partial(_edge_body, _K, nc, bpw, npad),
        out_type=(vec_t, vec_t, vec_t, vec_t),
        mesh=plsc.VectorSubcoreMesh(core_axis_name="c", subcore_axis_name="s"),
        compiler_params=pltpu.CompilerParams(needs_layout_passes=False),
        scratch_types=[
            pltpu.VMEM((npad,), f32),
            pltpu.VMEM((npad,), f32),
            pltpu.VMEM((npad,), f32),
            pltpu.VMEM((bpw,), jnp.int32),
            pltpu.VMEM((bpw,), f32),
            pltpu.VMEM((bpw,), f32),
            pltpu.VMEM((bpw,), f32),
            pltpu.VMEM((bpw,), f32),
        ],
    )(px, py, pz, src_pad)

    rows = bp // 128
    elen = pl.pallas_call(
        _len_body,
        grid=(1,),
        in_specs=[pl.BlockSpec((rows, 128), lambda i: (i, 0))],
        out_specs=pl.BlockSpec((rows, 128), lambda i: (i, 0)),
        out_shape=jax.ShapeDtypeStruct((rows, 128), jnp.float32),
    )(s2.reshape(rows, 128))

    dst = jnp.broadcast_to(
        jnp.arange(n, dtype=jnp.int32)[:, None], (n, _K)).reshape(-1)
    edge_index = jnp.stack([src, dst], axis=0)
    edge_vec = jnp.stack([vx[:nk], vy[:nk], vz[:nk]], axis=-1)
    edge_len = elen.reshape(-1)[:nk]
    return edge_index, edge_vec, edge_len


# speedup vs baseline: 2.0744x; 1.0050x over previous
import functools

import jax
import jax.numpy as jnp
from jax import lax
from jax.experimental import pallas as pl
from jax.experimental.pallas import tpu as pltpu
from jax.experimental.pallas import tpu_sc as plsc

_K = 19
_KP = 32
_R = 1024


def _round_up(x, m):
    return (x + m - 1) // m * m


def _knn_body(n, npad, r, pos_rows_ref, post_ref, idx_ref, d2_ref):
    i = pl.program_id(0)
    pr = pos_rows_ref[...]
    pc = post_ref[...]
    g = jnp.dot(pr, pc, preferred_element_type=jnp.float32)
    sq_r = (pr[:, 0:1] * pr[:, 0:1] + pr[:, 2:3] * pr[:, 2:3]) \
        + pr[:, 1:2] * pr[:, 1:2]
    sq_c = (pc[0:1, :] * pc[0:1, :] + pc[2:3, :] * pc[2:3, :]) \
        + pc[1:2, :] * pc[1:2, :]
    d2 = jnp.maximum(sq_r + sq_c - 2.0 * g, 0.0)
    col = lax.broadcasted_iota(jnp.int32, (r, npad), 1)
    row = i * r + lax.broadcasted_iota(jnp.int32, (r, npad), 0)
    inf = jnp.float32(jnp.inf)
    d2 = jnp.where((col == row) | (col >= n), inf, d2)
    d2_ref[...] = d2

    lane = lax.broadcasted_iota(jnp.int32, (r, _KP), 1)
    colf = col.astype(jnp.float32)
    big = jnp.float32(npad)

    def body(k, carry):
        idxs, prevf = carry
        d2v = d2_ref[...]
        d2v = jnp.where(colf == prevf, inf, d2v)
        d2_ref[...] = d2v
        m = jnp.min(d2v, axis=1, keepdims=True)
        idxf = jnp.min(jnp.where(d2v == m, colf, big), axis=1, keepdims=True)
        return jnp.where(lane == k, idxf.astype(jnp.int32), idxs), idxf

    init = (jnp.zeros((r, _KP), jnp.int32), jnp.full((r, 1), -1.0, jnp.float32))
    idx_ref[...] = lax.fori_loop(0, _K, body, init)[0]


def _edge_body(k, nc, bpw, npad,
               px_h, py_h, pz_h, src_h, vx_h, vy_h, vz_h, s2_h,
               px_v, py_v, pz_v, src_v, vx_v, vy_v, vz_v, s2_v):
    wid = lax.axis_index("s") * nc + lax.axis_index("c")
    base = wid * bpw
    pltpu.sync_copy(px_h, px_v)
    pltpu.sync_copy(py_h, py_v)
    pltpu.sync_copy(pz_h, pz_v)
    pltpu.sync_copy(src_h.at[pl.ds(base, bpw)], src_v)

    def body(j, carry):
        sl = pl.ds(j * 16, 16)
        sidx = src_v[sl]
        e = jnp.full((16,), base + j * 16, jnp.int32) + lax.iota(jnp.int32, 16)
        didx = lax.div(e, jnp.full((16,), k, jnp.int32))
        xs = plsc.load_gather(px_v, [sidx])
        ys = plsc.load_gather(py_v, [sidx])
        zs = plsc.load_gather(pz_v, [sidx])
        xd = plsc.load_gather(px_v, [didx])
        yd = plsc.load_gather(py_v, [didx])
        zd = plsc.load_gather(pz_v, [didx])
        dx = xs - xd
        dy = ys - yd
        dz = zs - zd
        vx_v[sl] = dx
        vy_v[sl] = dy
        vz_v[sl] = dz
        s2_v[sl] = (dx * dx + dz * dz) + dy * dy
        return carry

    lax.fori_loop(0, bpw // 16, body, 0)
    pltpu.sync_copy(vx_v, vx_h.at[pl.ds(base, bpw)])
    pltpu.sync_copy(vy_v, vy_h.at[pl.ds(base, bpw)])
    pltpu.sync_copy(vz_v, vz_h.at[pl.ds(base, bpw)])
    pltpu.sync_copy(s2_v, s2_h.at[pl.ds(base, bpw)])


def _len_body(s2_ref, len_ref):
    len_ref[...] = jnp.sqrt(s2_ref[...] + 1e-12)


@jax.jit
def kernel(pos):
    n = pos.shape[0]
    npad = _round_up(max(n, 128), 128)
    r = min(_R, _round_up(n, 8))
    nrow = _round_up(n, r)

    pos_rows = jnp.pad(pos, ((0, nrow - n), (0, 5)))
    post = jnp.pad(pos, ((0, npad - n), (0, 5))).T

    idxs = pl.pallas_call(
        functools.partial(_knn_body, n, npad, r),
        grid=(nrow // r,),
        in_specs=[
            pl.BlockSpec((r, 8), lambda i: (i, 0)),
            pl.BlockSpec((8, npad), lambda i: (0, 0)),
        ],
        out_specs=pl.BlockSpec((r, _KP), lambda i: (i, 0)),
        out_shape=jax.ShapeDtypeStruct((nrow, _KP), jnp.int32),
        scratch_shapes=[pltpu.VMEM((r, npad), jnp.float32)],
    )(pos_rows, post)

    src = idxs[:n, :_K].reshape(-1)
    nk = n * _K

    info = plsc.get_sparse_core_info()
    nc, ns = info.num_cores, info.num_subcores
    nw = nc * ns
    bp = _round_up(nk, 16 * nw)
    bpw = bp // nw

    src_pad = jnp.pad(src, (0, bp - nk))
    pos_cpad = jnp.pad(pos, ((0, npad - n), (0, 0)))
    px = pos_cpad[:, 0]
    py = pos_cpad[:, 1]
    pz = pos_cpad[:, 2]

    f32 = jnp.float32
    vec_t = jax.ShapeDtypeStruct((bp,), f32)
    vx, vy, vz, s2 = pl.kernel(
        functools.partial(_edge_body, _K, nc, bpw, npad),
        out_type=(vec_t, vec_t, vec_t, vec_t),
        mesh=plsc.VectorSubcoreMesh(core_axis_name="c", subcore_axis_name="s"),
        compiler_params=pltpu.CompilerParams(needs_layout_passes=False),
        scratch_types=[
            pltpu.VMEM((npad,), f32),
            pltpu.VMEM((npad,), f32),
            pltpu.VMEM((npad,), f32),
            pltpu.VMEM((bpw,), jnp.int32),
            pltpu.VMEM((bpw,), f32),
            pltpu.VMEM((bpw,), f32),
            pltpu.VMEM((bpw,), f32),
            pltpu.VMEM((bpw,), f32),
        ],
    )(px, py, pz, src_pad)

    rows = bp // 128
    elen = pl.pallas_call(
        _len_body,
        grid=(1,),
        in_specs=[pl.BlockSpec((rows, 128), lambda i: (i, 0))],
        out_specs=pl.BlockSpec((rows, 128), lambda i: (i, 0)),
        out_shape=jax.ShapeDtypeStruct((rows, 128), jnp.float32),
    )(s2.reshape(rows, 128))

    dst = jnp.broadcast_to(
        jnp.arange(n, dtype=jnp.int32)[:, None], (n, _K)).reshape(-1)
    edge_index = jnp.stack([src, dst], axis=0)
    edge_vec = jnp.stack([vx[:nk], vy[:nk], vz[:nk]], axis=-1)
    edge_len = elen.reshape(-1)[:nk]
    return edge_index, edge_vec, edge_len


# speedup vs baseline: 2.3039x; 1.1107x over previous
import functools

import jax
import jax.numpy as jnp
from jax import lax
from jax.experimental import pallas as pl
from jax.experimental.pallas import tpu as pltpu
from jax.experimental.pallas import tpu_sc as plsc

_K = 19
_KP = 32
_R = 1024


def _round_up(x, m):
    return (x + m - 1) // m * m


def _knn_body(n, npad, r, pos_rows_ref, post_ref, idx_ref, d2_ref):
    i = pl.program_id(0)
    pr = pos_rows_ref[...]
    pc = post_ref[...]
    g = jnp.dot(pr, pc, preferred_element_type=jnp.float32)
    sq_r = (pr[:, 0:1] * pr[:, 0:1] + pr[:, 2:3] * pr[:, 2:3]) \
        + pr[:, 1:2] * pr[:, 1:2]
    sq_c = (pc[0:1, :] * pc[0:1, :] + pc[2:3, :] * pc[2:3, :]) \
        + pc[1:2, :] * pc[1:2, :]
    d2 = jnp.maximum(sq_r + sq_c - 2.0 * g, 0.0)
    col = lax.broadcasted_iota(jnp.int32, (r, npad), 1)
    row = i * r + lax.broadcasted_iota(jnp.int32, (r, npad), 0)
    inf = jnp.float32(jnp.inf)
    d2 = jnp.where((col == row) | (col >= n), inf, d2)
    d2_ref[...] = d2

    lane = lax.broadcasted_iota(jnp.int32, (r, _KP), 1)
    colf = col.astype(jnp.float32)
    big = jnp.float32(npad)

    def body(k, carry):
        idxs, prevf = carry
        d2v = d2_ref[...]
        d2v = jnp.where(colf == prevf, inf, d2v)
        d2_ref[...] = d2v
        m = jnp.min(d2v, axis=1, keepdims=True)
        idxf = jnp.min(jnp.where(d2v == m, colf, big), axis=1, keepdims=True)
        return jnp.where(lane == k, idxf.astype(jnp.int32), idxs), idxf

    init = (jnp.zeros((r, _KP), jnp.int32), jnp.full((r, 1), -1.0, jnp.float32))
    idx_ref[...] = lax.fori_loop(0, _K, body, init, unroll=True)[0]


def _edge_body(k, nc, bpw, npad,
               px_h, py_h, pz_h, src_h, vx_h, vy_h, vz_h, s2_h,
               px_v, py_v, pz_v, src_v, vx_v, vy_v, vz_v, s2_v):
    wid = lax.axis_index("s") * nc + lax.axis_index("c")
    base = wid * bpw
    pltpu.sync_copy(px_h, px_v)
    pltpu.sync_copy(py_h, py_v)
    pltpu.sync_copy(pz_h, pz_v)
    pltpu.sync_copy(src_h.at[pl.ds(base, bpw)], src_v)

    def body(j, carry):
        sl = pl.ds(j * 16, 16)
        sidx = src_v[sl]
        e = jnp.full((16,), base + j * 16, jnp.int32) + lax.iota(jnp.int32, 16)
        didx = lax.div(e, jnp.full((16,), k, jnp.int32))
        xs = plsc.load_gather(px_v, [sidx])
        ys = plsc.load_gather(py_v, [sidx])
        zs = plsc.load_gather(pz_v, [sidx])
        xd = plsc.load_gather(px_v, [didx])
        yd = plsc.load_gather(py_v, [didx])
        zd = plsc.load_gather(pz_v, [didx])
        dx = xs - xd
        dy = ys - yd
        dz = zs - zd
        vx_v[sl] = dx
        vy_v[sl] = dy
        vz_v[sl] = dz
        s2_v[sl] = (dx * dx + dz * dz) + dy * dy
        return carry

    lax.fori_loop(0, bpw // 16, body, 0)
    pltpu.sync_copy(vx_v, vx_h.at[pl.ds(base, bpw)])
    pltpu.sync_copy(vy_v, vy_h.at[pl.ds(base, bpw)])
    pltpu.sync_copy(vz_v, vz_h.at[pl.ds(base, bpw)])
    pltpu.sync_copy(s2_v, s2_h.at[pl.ds(base, bpw)])


def _len_body(s2_ref, len_ref):
    len_ref[...] = jnp.sqrt(s2_ref[...] + 1e-12)


@jax.jit
def kernel(pos):
    n = pos.shape[0]
    npad = _round_up(max(n, 128), 128)
    r = min(_R, _round_up(n, 8))
    nrow = _round_up(n, r)

    pos_rows = jnp.pad(pos, ((0, nrow - n), (0, 5)))
    post = jnp.pad(pos, ((0, npad - n), (0, 5))).T

    idxs = pl.pallas_call(
        functools.partial(_knn_body, n, npad, r),
        grid=(nrow // r,),
        in_specs=[
            pl.BlockSpec((r, 8), lambda i: (i, 0)),
            pl.BlockSpec((8, npad), lambda i: (0, 0)),
        ],
        out_specs=pl.BlockSpec((r, _KP), lambda i: (i, 0)),
        out_shape=jax.ShapeDtypeStruct((nrow, _KP), jnp.int32),
        scratch_shapes=[pltpu.VMEM((r, npad), jnp.float32)],
    )(pos_rows, post)

    src = idxs[:n, :_K].reshape(-1)
    nk = n * _K

    info = plsc.get_sparse_core_info()
    nc, ns = info.num_cores, info.num_subcores
    nw = nc * ns
    bp = _round_up(nk, 16 * nw)
    bpw = bp // nw

    src_pad = jnp.pad(src, (0, bp - nk))
    pos_cpad = jnp.pad(pos, ((0, npad - n), (0, 0)))
    px = pos_cpad[:, 0]
    py = pos_cpad[:, 1]
    pz = pos_cpad[:, 2]

    f32 = jnp.float32
    vec_t = jax.ShapeDtypeStruct((bp,), f32)
    vx, vy, vz, s2 = pl.kernel(
        functools.partial(_edge_body, _K, nc, bpw, npad),
        out_type=(vec_t, vec_t, vec_t, vec_t),
        mesh=plsc.VectorSubcoreMesh(core_axis_name="c", subcore_axis_name="s"),
        compiler_params=pltpu.CompilerParams(needs_layout_passes=False),
        scratch_types=[
            pltpu.VMEM((npad,), f32),
            pltpu.VMEM((npad,), f32),
            pltpu.VMEM((npad,), f32),
            pltpu.VMEM((bpw,), jnp.int32),
            pltpu.VMEM((bpw,), f32),
            pltpu.VMEM((bpw,), f32),
            pltpu.VMEM((bpw,), f32),
            pltpu.VMEM((bpw,), f32),
        ],
    )(px, py, pz, src_pad)

    rows = bp // 128
    elen = pl.pallas_call(
        _len_body,
        grid=(1,),
        in_specs=[pl.BlockSpec((rows, 128), lambda i: (i, 0))],
        out_specs=pl.BlockSpec((rows, 128), lambda i: (i, 0)),
        out_shape=jax.ShapeDtypeStruct((rows, 128), jnp.float32),
    )(s2.reshape(rows, 128))

    dst = jnp.broadcast_to(
        jnp.arange(n, dtype=jnp.int32)[:, None], (n, _K)).reshape(-1)
    edge_index = jnp.stack([src, dst], axis=0)
    edge_vec = jnp.stack([vx[:nk], vy[:nk], vz[:nk]], axis=-1)
    edge_len = elen.reshape(-1)[:nk]
    return edge_index, edge_vec, edge_len
